# baseline (speedup 1.0000x reference)
import functools

import jax
import jax.numpy as jnp
from jax import lax
from jax.experimental import pallas as pl
from jax.experimental.pallas import tpu as pltpu
from jax.experimental.pallas import tpu_sc as plsc

N_UNITS = 64
B_ROWS = 4096
SEQ = 200
NC = 2
NS = 16
NW = NC * NS
BPW = B_ROWS // NW
NGBUF = 4
NPBUF = 2
LOOKAHEAD = 2
SCALE = 8.0

_mesh = plsc.VectorSubcoreMesh(core_axis_name="c", subcore_axis_name="s")


@functools.partial(
    pl.kernel,
    mesh=_mesh,
    out_type=jax.ShapeDtypeStruct((SEQ, 8, NW, 8, 128), jnp.float32),
    scratch_types=[
        pltpu.VMEM((BPW, SEQ), jnp.int32),
        pltpu.VMEM((SEQ, BPW), jnp.int32),
        pltpu.VMEM((NGBUF, BPW, N_UNITS), jnp.float32),
        pltpu.VMEM((NPBUF, N_UNITS, BPW), jnp.float32),
        pltpu.SemaphoreType.DMA((NGBUF,)),
        pltpu.SemaphoreType.DMA((NPBUF,)),
    ],
    compiler_params=pltpu.CompilerParams(use_tc_tiling_on_sc=False,
                                         needs_layout_passes=False),
)
def _emb_lookup(x_hbm, table_hbm, out_hbm, idx_v, idx_t, gbuf, pbuf,
                gsem, psem):
    wid = lax.axis_index("s") * NC + lax.axis_index("c")

    pltpu.sync_copy(x_hbm.at[pl.ds(wid * BPW, BPW)], idx_v)

    iota16 = lax.iota(jnp.int32, 16)
    tail_mask = iota16 >= 8

    def tr_row(r, carry):
        col = jnp.full((16,), r, jnp.int32)
        for c in range(12):
            plsc.store_scatter(idx_t, [iota16 + (16 * c), col],
                               idx_v[r, pl.ds(16 * c, 16)])
        plsc.store_scatter(idx_t, [iota16 + 184, col],
                           idx_v[r, pl.ds(184, 16)], mask=tail_mask)
        return carry

    lax.fori_loop(0, BPW, tr_row, 0)

    def fire_gather(s, g):
        pltpu.async_copy(table_hbm.at[idx_t.at[s]], gbuf.at[g], gsem.at[g])

    for s0 in range(LOOKAHEAD):
        fire_gather(s0, s0)

    row_sel = [iota16 + (16 * o) for o in range(4)]

    def group(h, carry):
        for b in range(NGBUF):
            s = h * NGBUF + b
            p = b % NPBUF
            pltpu.make_async_copy(table_hbm.at[pl.ds(0, BPW)], gbuf.at[b],
                                  gsem.at[b]).wait()

            @pl.when(s >= NPBUF)
            def _drain():
                for t in range(8):
                    pltpu.make_async_copy(out_hbm.at[0, 0, 0],
                                          pbuf.at[p, pl.ds(8 * t, 8)],
                                          psem.at[p]).wait()

            def tpose(j, c2, _b=b, _p=p):
                colj = jnp.full((16,), j, jnp.int32)
                for o in range(4):
                    plsc.store_scatter(
                        pbuf.at[_p], [row_sel[o], colj],
                        gbuf[_b, j, pl.ds(16 * o, 16)] * SCALE)
                return c2

            lax.fori_loop(0, BPW, tpose, 0)

            for t in range(8):
                pltpu.async_copy(pbuf.at[p, pl.ds(8 * t, 8)],
                                 out_hbm.at[s, t, wid], psem.at[p])

            @pl.when(s + LOOKAHEAD < SEQ)
            def _refill():
                fire_gather(s + LOOKAHEAD, (b + LOOKAHEAD) % NGBUF)

        return carry

    lax.fori_loop(0, SEQ // NGBUF, group, 0)

    for p in range(NPBUF):
        for t in range(8):
            pltpu.make_async_copy(out_hbm.at[0, 0, 0],
                                  pbuf.at[p, pl.ds(8 * t, 8)],
                                  psem.at[p]).wait()


def kernel(x, table):
    o5 = _emb_lookup(x, table)
    return o5.transpose(2, 4, 0, 1, 3).reshape(B_ROWS, SEQ, N_UNITS)


# speedup vs baseline: 1.3139x; 1.3139x over previous
import functools

import jax
import jax.numpy as jnp
from jax import lax
from jax.experimental import pallas as pl
from jax.experimental.pallas import tpu as pltpu
from jax.experimental.pallas import tpu_sc as plsc

N_UNITS = 64
B_ROWS = 4096
SEQ = 200
NC = 2
NS = 16
NW = NC * NS
BPW = B_ROWS // NW
NGBUF = 4
NPBUF = 2
LOOKAHEAD = 2
SCALE = 8.0

_mesh = plsc.VectorSubcoreMesh(core_axis_name="c", subcore_axis_name="s")


@functools.partial(
    pl.kernel,
    mesh=_mesh,
    out_type=jax.ShapeDtypeStruct((SEQ, 8, NW, 8, 128), jnp.float32),
    scratch_types=[
        pltpu.VMEM((BPW, SEQ), jnp.int32),
        pltpu.VMEM((SEQ, BPW), jnp.int32),
        pltpu.VMEM((NGBUF, BPW, N_UNITS), jnp.float32),
        pltpu.VMEM((NPBUF, N_UNITS, BPW), jnp.float32),
        pltpu.SemaphoreType.DMA((NGBUF,)),
        pltpu.SemaphoreType.DMA((NPBUF,)),
    ],
    compiler_params=pltpu.CompilerParams(use_tc_tiling_on_sc=False,
                                         needs_layout_passes=False),
)
def _emb_lookup(x_hbm, table_hbm, out_hbm, idx_v, idx_t, gbuf, pbuf,
                gsem, psem):
    wid = lax.axis_index("s") * NC + lax.axis_index("c")

    pltpu.sync_copy(x_hbm.at[pl.ds(wid * BPW, BPW)], idx_v)

    iota16 = lax.iota(jnp.int32, 16)
    tail_mask = iota16 >= 8

    @plsc.parallel_loop(0, BPW, unroll=4)
    def tr_row(r):
        col = jnp.full((16,), r, jnp.int32)
        for c in range(12):
            plsc.store_scatter(idx_t, [iota16 + (16 * c), col],
                               idx_v[r, pl.ds(16 * c, 16)])
        plsc.store_scatter(idx_t, [iota16 + 184, col],
                           idx_v[r, pl.ds(184, 16)], mask=tail_mask)

    def fire_gather(s, g):
        pltpu.async_copy(table_hbm.at[idx_t.at[s]], gbuf.at[g], gsem.at[g])

    for s0 in range(LOOKAHEAD):
        fire_gather(s0, s0)

    row_sel = [iota16 + (16 * o) for o in range(4)]

    def group(h, carry):
        for b in range(NGBUF):
            s = h * NGBUF + b
            p = b % NPBUF
            pltpu.make_async_copy(table_hbm.at[pl.ds(0, BPW)], gbuf.at[b],
                                  gsem.at[b]).wait()

            @pl.when(s >= NPBUF)
            def _drain():
                for t in range(8):
                    pltpu.make_async_copy(out_hbm.at[0, 0, 0],
                                          pbuf.at[p, pl.ds(8 * t, 8)],
                                          psem.at[p]).wait()

            @plsc.parallel_loop(0, BPW, unroll=8)
            def tpose(j, _b=b, _p=p):
                colj = jnp.full((16,), j, jnp.int32)
                for o in range(4):
                    plsc.store_scatter(
                        pbuf.at[_p], [row_sel[o], colj],
                        gbuf[_b, j, pl.ds(16 * o, 16)] * SCALE)

            for t in range(8):
                pltpu.async_copy(pbuf.at[p, pl.ds(8 * t, 8)],
                                 out_hbm.at[s, t, wid], psem.at[p])

            @pl.when(s + LOOKAHEAD < SEQ)
            def _refill():
                fire_gather(s + LOOKAHEAD, (b + LOOKAHEAD) % NGBUF)

        return carry

    lax.fori_loop(0, SEQ // NGBUF, group, 0)

    for p in range(NPBUF):
        for t in range(8):
            pltpu.make_async_copy(out_hbm.at[0, 0, 0],
                                  pbuf.at[p, pl.ds(8 * t, 8)],
                                  psem.at[p]).wait()


def kernel(x, table):
    o5 = _emb_lookup(x, table)
    return o5.transpose(2, 4, 0, 1, 3).reshape(B_ROWS, SEQ, N_UNITS)


# speedup vs baseline: 1.4265x; 1.0857x over previous
import functools

import jax
import jax.numpy as jnp
from jax import lax
from jax.experimental import pallas as pl
from jax.experimental.pallas import tpu as pltpu
from jax.experimental.pallas import tpu_sc as plsc

N_UNITS = 64
CHUNK = 128
B_TOTAL = 4096 * 200
NC = 2
NS = 16
NW = NC * NS
IDX_PER_W = B_TOTAL // NW
N_CHUNKS = IDX_PER_W // CHUNK
NBUF = 8
LOOKAHEAD = 4
SCALE = 8.0

_mesh = plsc.VectorSubcoreMesh(core_axis_name="c", subcore_axis_name="s")


@functools.partial(
    pl.kernel,
    mesh=_mesh,
    out_type=jax.ShapeDtypeStruct((B_TOTAL, N_UNITS), jnp.float32),
    scratch_types=[
        pltpu.VMEM((N_CHUNKS, CHUNK), jnp.int32),
        pltpu.VMEM((NBUF, CHUNK, N_UNITS), jnp.float32),
        pltpu.SemaphoreType.DMA((NBUF,)),
        pltpu.SemaphoreType.DMA((NBUF,)),
    ],
    compiler_params=pltpu.CompilerParams(use_tc_tiling_on_sc=False),
)
def _emb_lookup(x_hbm, table_hbm, out_hbm, idx_v, bufs, gsem, osem):
    wid = lax.axis_index("s") * NC + lax.axis_index("c")
    idx_base = wid * IDX_PER_W

    pltpu.sync_copy(x_hbm.at[pl.ds(wid * N_CHUNKS, N_CHUNKS)], idx_v)

    def fire_gather(chunk_i, b):
        pltpu.async_copy(table_hbm.at[idx_v.at[chunk_i]], bufs.at[b],
                         gsem.at[b])

    def out_slice(chunk_i):
        return out_hbm.at[pl.ds((idx_base + chunk_i * CHUNK), CHUNK)]

    for b in range(LOOKAHEAD):
        fire_gather(b, b)

    def group(h, carry):
        for b in range(NBUF):
            i = h * NBUF + b
            pltpu.make_async_copy(bufs.at[b], out_slice(i), gsem.at[b]).wait()

            def scale(r, c2, _b=b):
                for rr in range(2):
                    for o in range(0, N_UNITS, 16):
                        bufs[_b, r * 2 + rr, pl.ds(o, 16)] = (
                            bufs[_b, r * 2 + rr, pl.ds(o, 16)] * SCALE)
                return c2

            lax.fori_loop(0, CHUNK // 2, scale, 0)

            pltpu.async_copy(bufs.at[b], out_slice(i), osem.at[b])

            bq = (b + LOOKAHEAD) % NBUF

            @pl.when(i >= LOOKAHEAD)
            def _drain():
                pltpu.make_async_copy(bufs.at[bq], out_slice(0),
                                      osem.at[bq]).wait()

            @pl.when(i + LOOKAHEAD < N_CHUNKS)
            def _refill():
                fire_gather(i + LOOKAHEAD, bq)

        return carry

    lax.fori_loop(0, N_CHUNKS // NBUF, group, 0)

    for b in range(LOOKAHEAD, NBUF):
        pltpu.make_async_copy(bufs.at[b], out_slice(0), osem.at[b]).wait()


def kernel(x, table):
    xf = x.reshape(B_TOTAL // CHUNK, CHUNK)
    out = _emb_lookup(xf, table)
    return out.reshape(x.shape + (N_UNITS,))
